# Initial kernel scaffold; baseline (speedup 1.0000x reference)
import jax
import jax.numpy as jnp
from jax.experimental import pallas as pl


def kernel(edge_list, edge_weight, edge_feature, num_node, node_position, x, W_in, b_in, W_rbf, b_rbf, W_edge, b_edge, W_out, b_out):
    raise NotImplementedError("write your pallas kernel here")


# speedup vs baseline: 1.8922x; 1.8922x over previous
import functools
import math

import jax
import jax.numpy as jnp
from jax import lax
from jax.experimental import pallas as pl
from jax.experimental.pallas import tpu as pltpu
from jax.experimental.pallas import tpu_sc as plsc

N_NODES = 10000
D_HID = 256
QRT = 64
K_RBF = 100
K_PAD = 128
CUTOFF = 5.0
LOG2 = math.log(2.0)

NC, NS, L = 2, 16, 16
BATCH = 128
E_PAD = 161792
NB = E_PAD // (NS * BATCH)
EPW_A = E_PAD // (NC * NS)
EPS_C = E_PAD // NS
N_ACC = 10240
RPW = N_ACC // NS
EB = 1024

_mesh = plsc.VectorSubcoreMesh(
    core_axis_name="c", subcore_axis_name="s", num_cores=NC, num_subcores=NS)


@functools.partial(
    pl.kernel,
    out_type=jax.ShapeDtypeStruct((E_PAD,), jnp.float32),
    mesh=_mesh,
    compiler_params=pltpu.CompilerParams(needs_layout_passes=False),
    scratch_types=[
        pltpu.VMEM((EPW_A,), jnp.int32),
        pltpu.VMEM((EPW_A,), jnp.int32),
        pltpu.VMEM((N_NODES,), jnp.float32),
        pltpu.VMEM((N_NODES,), jnp.float32),
        pltpu.VMEM((N_NODES,), jnp.float32),
        pltpu.VMEM((EPW_A,), jnp.float32),
    ],
)
def _dist2_sc(src_hbm, dst_hbm, px_hbm, py_hbm, pz_hbm, d2_hbm,
              src_v, dst_v, px_v, py_v, pz_v, d2_v):
    c = lax.axis_index("c")
    s = lax.axis_index("s")
    base = (s * NC + c) * EPW_A
    pltpu.sync_copy(px_hbm, px_v)
    pltpu.sync_copy(py_hbm, py_v)
    pltpu.sync_copy(pz_hbm, pz_v)
    pltpu.sync_copy(src_hbm.at[pl.ds(base, EPW_A)], src_v)
    pltpu.sync_copy(dst_hbm.at[pl.ds(base, EPW_A)], dst_v)

    def body(j, carry):
        sl = pl.ds(j * L, L)
        si = src_v[sl]
        di = dst_v[sl]
        dx = plsc.load_gather(px_v, [si]) - plsc.load_gather(px_v, [di])
        dy = plsc.load_gather(py_v, [si]) - plsc.load_gather(py_v, [di])
        dz = plsc.load_gather(pz_v, [si]) - plsc.load_gather(pz_v, [di])
        d2_v[sl] = dx * dx + dy * dy + dz * dz
        return carry

    lax.fori_loop(0, EPW_A // L, body, 0)
    pltpu.sync_copy(d2_v, d2_hbm.at[pl.ds(base, EPW_A)])


def _h_tc_body(x_ref, wT_ref, b_ref, out_ref):
    hb = jnp.dot(x_ref[...], wT_ref[...],
                 preferred_element_type=jnp.float32) + b_ref[...]
    out_ref[0, :, :] = hb[:, :2 * QRT]
    out_ref[1, :, :] = hb[:, 2 * QRT:]


def _edge_tc_body(d2_ref, ew_ref, ef_ref, hs_ref, cen_ref, wrbfT_ref,
                  brbf_ref, wedgeT_ref, bedge_ref, out_ref):
    d = jnp.sqrt(d2_ref[...])
    t = d - cen_ref[...]
    rb = jnp.exp(-(t * t))
    rw = jnp.dot(rb, wrbfT_ref[...],
                 preferred_element_type=jnp.float32) + brbf_ref[...]
    cc = ew_ref[...] * rw
    eh = jnp.dot(ef_ref[...], wedgeT_ref[...],
                 preferred_element_type=jnp.float32) + bedge_ref[...]
    hs = jnp.concatenate([hs_ref[0], hs_ref[1]], axis=1)
    msg = cc * (hs + eh)
    out_ref[0, :, :] = msg[:, :2 * QRT]
    out_ref[1, :, :] = msg[:, 2 * QRT:]


def _out_tc_body(u_ref, woT_ref, b_ref, out_ref):
    o = jnp.dot(u_ref[...], woT_ref[...],
                preferred_element_type=jnp.float32) + b_ref[...]
    out_ref[...] = (jnp.maximum(o, 0.0)
                    + jnp.log1p(jnp.exp(-jnp.abs(o))) - LOG2)


@functools.partial(
    pl.kernel,
    out_type=jax.ShapeDtypeStruct((2 * E_PAD, 2 * QRT), jnp.float32),
    mesh=_mesh,
    compiler_params=pltpu.CompilerParams(needs_layout_passes=False),
    scratch_types=[
        pltpu.VMEM((NB, BATCH), jnp.int32),
        pltpu.VMEM((BATCH, 2 * QRT), jnp.float32),
        pltpu.SemaphoreType.DMA,
    ],
)
def _gather_sc(src6_hbm, h2_hbm, hs_hbm, src_m, rows_v, sem):
    c = lax.axis_index("c")
    s = lax.axis_index("s")
    pltpu.sync_copy(src6_hbm.at[c * NS + s], src_m)
    hoff = c * E_PAD + s * EPS_C

    def batch(b, carry):
        pltpu.async_copy(h2_hbm.at[src_m.at[b]], rows_v, sem).wait()
        pltpu.sync_copy(rows_v, hs_hbm.at[pl.ds(hoff + b * BATCH, BATCH)])
        return carry

    lax.fori_loop(0, NB, batch, 0)


def kernel(edge_list, edge_weight, edge_feature, num_node, node_position, x,
           W_in, b_in, W_rbf, b_rbf, W_edge, b_edge, W_out, b_out):
    del num_node
    E = edge_list.shape[0]
    pad = E_PAD - E
    src = jnp.pad(edge_list[:, 0], (0, pad))
    dst = jnp.pad(edge_list[:, 1], (0, pad))
    ew = jnp.pad(edge_weight, (0, pad))
    ef = jnp.pad(edge_feature, ((0, pad), (0, 0)))

    px = node_position[:, 0]
    py = node_position[:, 1]
    pz = node_position[:, 2]

    d2 = _dist2_sc(src, dst, px, py, pz)

    n_rows = x.shape[0]
    rb_blk = 2000
    h2 = pl.pallas_call(
        _h_tc_body,
        grid=(n_rows // rb_blk,),
        in_specs=[
            pl.BlockSpec((rb_blk, D_HID), lambda i: (i, 0)),
            pl.BlockSpec((D_HID, D_HID), lambda i: (0, 0)),
            pl.BlockSpec((1, D_HID), lambda i: (0, 0)),
        ],
        out_specs=pl.BlockSpec((2, rb_blk, 2 * QRT), lambda i: (0, i, 0)),
        out_shape=jax.ShapeDtypeStruct((2, n_rows, 2 * QRT), jnp.float32),
    )(x, W_in.T, b_in.reshape(1, D_HID))

    centers = jnp.concatenate(
        [jnp.linspace(0.0, CUTOFF, K_RBF, dtype=jnp.float32),
         jnp.full((K_PAD - K_RBF,), 1e6, dtype=jnp.float32)]).reshape(1, K_PAD)
    wrbfT = jnp.zeros((K_PAD, D_HID), jnp.float32).at[:K_RBF].set(W_rbf.T)

    src3 = src.reshape(NS, NB, BATCH)
    src6 = jnp.concatenate([src3, src3 + n_rows], axis=0)
    hs = _gather_sc(src6, h2.reshape(2 * n_rows, 2 * QRT))

    M = pl.pallas_call(
        _edge_tc_body,
        grid=(E_PAD // EB,),
        in_specs=[
            pl.BlockSpec((EB, 1), lambda i: (i, 0)),
            pl.BlockSpec((EB, 1), lambda i: (i, 0)),
            pl.BlockSpec((EB, 16), lambda i: (i, 0)),
            pl.BlockSpec((2, EB, 2 * QRT), lambda i: (0, i, 0)),
            pl.BlockSpec((1, K_PAD), lambda i: (0, 0)),
            pl.BlockSpec((K_PAD, D_HID), lambda i: (0, 0)),
            pl.BlockSpec((1, D_HID), lambda i: (0, 0)),
            pl.BlockSpec((16, D_HID), lambda i: (0, 0)),
            pl.BlockSpec((1, D_HID), lambda i: (0, 0)),
        ],
        out_specs=pl.BlockSpec((2, EB, 2 * QRT), lambda i: (0, i, 0)),
        out_shape=jax.ShapeDtypeStruct((2, E_PAD, 2 * QRT), jnp.float32),
    )(d2.reshape(E_PAD, 1), ew.reshape(E_PAD, 1), ef,
      hs.reshape(2, E_PAD, 2 * QRT), centers, wrbfT,
      b_rbf.reshape(1, D_HID), W_edge.T, b_edge.reshape(1, D_HID))

    msg = jnp.concatenate([M[0], M[1]], axis=1)
    upd = jax.ops.segment_sum(msg, dst, num_segments=n_rows)

    out = pl.pallas_call(
        _out_tc_body,
        grid=(n_rows // rb_blk,),
        in_specs=[
            pl.BlockSpec((rb_blk, D_HID), lambda i: (i, 0)),
            pl.BlockSpec((D_HID, D_HID), lambda i: (0, 0)),
            pl.BlockSpec((1, D_HID), lambda i: (0, 0)),
        ],
        out_specs=pl.BlockSpec((rb_blk, D_HID), lambda i: (i, 0)),
        out_shape=jax.ShapeDtypeStruct((n_rows, D_HID), jnp.float32),
    )(upd, W_out.T, b_out.reshape(1, D_HID))
    return out
